# baseline (speedup 1.0000x reference)
import jax
import jax.numpy as jnp
from jax.experimental import pallas as pl
from jax.experimental.pallas import tpu as pltpu

TOKENS = 16384
D_MODEL = 2048
HIDDEN = 256
N_EXPERTS = 64
TOP_K = 8
BT = 2048
NBLK = TOKENS // BT
NX = 2
DH = D_MODEL // NX
RB = 256


def _gating_kernel(xa_ref, xb_ref, w1_ref, b1_ref, w2_ref, b2_ref, out_ref,
                   logits_ref):
    i = pl.program_id(0)

    @pl.when(i < NBLK)
    def _compute_logits():
        acc = jnp.dot(xa_ref[...], w1_ref[:DH, :],
                      preferred_element_type=jnp.float32)
        acc += jnp.dot(xb_ref[...], w1_ref[DH:, :],
                       preferred_element_type=jnp.float32)
        h = acc + b1_ref[...]
        h = 0.5 * h * (1.0 + jax.lax.erf(h * 0.7071067811865476))
        logits_ref[i % 2, :, :] = (
            jnp.dot(h, w2_ref[...], preferred_element_type=jnp.float32)
            + b2_ref[...]
        )

    @pl.when(i > 0)
    def _route():
        slot = (i - 1) % 2
        for r in range(BT // RB):
            logits = logits_ref[slot, r * RB:(r + 1) * RB, :]

            t = jnp.max(logits, axis=-1, keepdims=True)
            mx = t
            for _ in range(TOP_K - 1):
                t = jnp.max(
                    jnp.where(logits >= t, -jnp.inf, logits),
                    axis=-1,
                    keepdims=True,
                )

            e = jnp.where(logits >= t, jnp.exp(logits - mx), 0.0)
            z = jnp.sum(e, axis=-1, keepdims=True)
            out_ref[pl.ds(r * RB, RB), :] = e * (1.0 / z)


@jax.jit
def kernel(x, W1, b1, W2, b2):
    w1t = W1.T
    w2t = W2.T
    b1r = b1.reshape(1, HIDDEN)
    b2r = b2.reshape(1, N_EXPERTS)

    grid = (NBLK + 1,)
    sparse_weights = pl.pallas_call(
        _gating_kernel,
        grid=grid,
        in_specs=[
            pl.BlockSpec((BT, DH), lambda i: (jnp.minimum(i, NBLK - 1), 0)),
            pl.BlockSpec((BT, DH), lambda i: (jnp.minimum(i, NBLK - 1), 1)),
            pl.BlockSpec((D_MODEL, HIDDEN), lambda i: (0, 0)),
            pl.BlockSpec((1, HIDDEN), lambda i: (0, 0)),
            pl.BlockSpec((HIDDEN, N_EXPERTS), lambda i: (0, 0)),
            pl.BlockSpec((1, N_EXPERTS), lambda i: (0, 0)),
        ],
        out_specs=pl.BlockSpec(
            (BT, N_EXPERTS), lambda i: (jnp.maximum(i - 1, 0), 0)
        ),
        out_shape=jax.ShapeDtypeStruct((TOKENS, N_EXPERTS), jnp.float32),
        scratch_shapes=[pltpu.VMEM((2, BT, N_EXPERTS), jnp.float32)],
        compiler_params=pltpu.CompilerParams(
            dimension_semantics=("arbitrary",),
        ),
    )(x, x, w1t, b1r, w2t, b2r)

    aux_loss = jnp.asarray(0.0, dtype=jnp.float32)
    return (sparse_weights, aux_loss)


# speedup vs baseline: 1.2766x; 1.2766x over previous
import jax
import jax.numpy as jnp
from jax.experimental import pallas as pl
from jax.experimental.pallas import tpu as pltpu

TOKENS = 16384
D_MODEL = 2048
HIDDEN = 256
N_EXPERTS = 64
TOP_K = 8
BT = 2048
NX = 2
DH = D_MODEL // NX
RB = 256


def _gating_kernel(xa_ref, xb_ref, w1_ref, b1_ref, w2_ref,
                   b2_ref, out_ref):
    x_refs = (xa_ref, xb_ref)
    acc = None
    for k, xr in enumerate(x_refs):
        part = jnp.dot(
            xr[...],
            w1_ref[k * DH:(k + 1) * DH, :],
            preferred_element_type=jnp.float32,
        )
        acc = part if acc is None else acc + part
    h = acc + b1_ref[...]
    h = 0.5 * h * (1.0 + jax.lax.erf(h * 0.7071067811865476))
    all_logits = (
        jnp.dot(h, w2_ref[...], preferred_element_type=jnp.float32)
        + b2_ref[...]
    )

    for r in range(BT // RB):
        rows = pl.ds(r * RB, RB)
        logits = all_logits[r * RB:(r + 1) * RB, :]

        t = jnp.max(logits, axis=-1, keepdims=True)
        mx = t
        for _ in range(TOP_K - 1):
            t = jnp.max(
                jnp.where(logits >= t, -jnp.inf, logits),
                axis=-1,
                keepdims=True,
            )

        e = jnp.where(logits >= t, jnp.exp(logits - mx), 0.0)
        z = jnp.sum(e, axis=-1, keepdims=True)
        out_ref[rows, :] = e * (1.0 / z)


@jax.jit
def kernel(x, W1, b1, W2, b2):
    w1t = W1.T
    w2t = W2.T
    b1r = b1.reshape(1, HIDDEN)
    b2r = b2.reshape(1, N_EXPERTS)

    grid = (TOKENS // BT,)
    sparse_weights = pl.pallas_call(
        _gating_kernel,
        grid=grid,
        in_specs=[
            pl.BlockSpec((BT, DH), lambda i: (i, 0)),
            pl.BlockSpec((BT, DH), lambda i: (i, 1)),
            pl.BlockSpec((D_MODEL, HIDDEN), lambda i: (0, 0)),
            pl.BlockSpec((1, HIDDEN), lambda i: (0, 0)),
            pl.BlockSpec((HIDDEN, N_EXPERTS), lambda i: (0, 0)),
            pl.BlockSpec((1, N_EXPERTS), lambda i: (0, 0)),
        ],
        out_specs=pl.BlockSpec((BT, N_EXPERTS), lambda i: (i, 0)),
        out_shape=jax.ShapeDtypeStruct((TOKENS, N_EXPERTS), jnp.float32),
        compiler_params=pltpu.CompilerParams(
            dimension_semantics=("parallel",),
        ),
    )(x, x, w1t, b1r, w2t, b2r)

    aux_loss = jnp.asarray(0.0, dtype=jnp.float32)
    return (sparse_weights, aux_loss)
